# Initial kernel scaffold; baseline (speedup 1.0000x reference)
import jax
import jax.numpy as jnp
from jax.experimental import pallas as pl


def kernel(x):
    raise NotImplementedError("write your pallas kernel here")


# speedup vs baseline: 6.5804x; 6.5804x over previous
import jax
import jax.numpy as jnp
from jax.experimental import pallas as pl

INPUT_DIM = 512
TIME_STEPS = 100
MAX_LAT = 99
BT = 8


def _body(x_ref, o_ref):
    xv = x_ref[...]
    s = jax.nn.sigmoid(xv)
    lat = jnp.clip(jnp.round((1.0 - s) * float(MAX_LAT)).astype(jnp.int32), 0, MAX_LAT)
    val = jnp.where(s > 0.5, 1.0, 0.0).astype(jnp.float32)
    t = jax.lax.broadcasted_iota(jnp.int32, (BT, INPUT_DIM, TIME_STEPS), 2)
    o_ref[...] = jnp.where(t == lat[:, :, None], val[:, :, None], 0.0)


def kernel(x):
    B, F = x.shape
    return pl.pallas_call(
        _body,
        grid=(B // BT,),
        in_specs=[pl.BlockSpec((BT, F), lambda i: (i, 0))],
        out_specs=pl.BlockSpec((BT, F, TIME_STEPS), lambda i: (i, 0, 0)),
        out_shape=jax.ShapeDtypeStruct((B, F, TIME_STEPS), jnp.float32),
    )(x)


# speedup vs baseline: 6.5948x; 1.0022x over previous
import jax
import jax.numpy as jnp
from jax import lax
from jax.experimental import pallas as pl

INPUT_DIM = 512
TIME_STEPS = 100
MAX_LAT = 99
BT = 8
NLANE = 128


def _body(xt_ref, o_ref):
    xv = xt_ref[0]
    s = jax.nn.sigmoid(xv)
    latf = jnp.clip(jnp.round((1.0 - s) * float(MAX_LAT)), 0.0, float(MAX_LAT))
    bsel = lax.broadcasted_iota(jnp.int32, (BT, BT * NLANE), 1) // NLANE
    wsel = jnp.where(bsel == lax.broadcasted_iota(jnp.int32, (BT, BT * NLANE), 0),
                     1.0, 0.0).astype(jnp.bfloat16)
    latb = lax.dot_general(latf.astype(jnp.bfloat16), wsel,
                           (((1,), (0,)), ((), ())),
                           preferred_element_type=jnp.float32)
    t = lax.broadcasted_iota(jnp.int32, (INPUT_DIM, BT * NLANE), 1) % NLANE
    hit = (latb == t.astype(jnp.float32)) & (t <= (MAX_LAT - 1) // 2)
    oneh = jnp.where(hit, 1.0, 0.0).astype(jnp.float32)
    for b in range(BT):
        o_ref[b] = oneh[:, b * NLANE:b * NLANE + TIME_STEPS]


def kernel(x):
    B, F = x.shape
    xt = x.reshape(B // BT, BT, F).transpose(0, 2, 1)
    return pl.pallas_call(
        _body,
        grid=(B // BT,),
        in_specs=[pl.BlockSpec((1, F, BT), lambda i: (i, 0, 0))],
        out_specs=pl.BlockSpec((BT, F, TIME_STEPS), lambda i: (i, 0, 0)),
        out_shape=jax.ShapeDtypeStruct((B, F, TIME_STEPS), jnp.float32),
    )(xt)


# speedup vs baseline: 7.2287x; 1.0961x over previous
import jax
import jax.numpy as jnp
from jax import lax
from jax.experimental import pallas as pl

INPUT_DIM = 512
TIME_STEPS = 100
MAX_LAT = 99
BT = 8
NLANE = 128


def _body(x_ref, o_ref):
    xv = x_ref[...]
    s = jax.nn.sigmoid(xv)
    latf = jnp.clip(jnp.round((1.0 - s) * float(MAX_LAT)), 0.0, float(MAX_LAT))
    bsel = lax.broadcasted_iota(jnp.int32, (BT, BT * NLANE), 1) // NLANE
    wsel = jnp.where(bsel == lax.broadcasted_iota(jnp.int32, (BT, BT * NLANE), 0),
                     1.0, 0.0).astype(jnp.bfloat16)
    latb = lax.dot_general(latf.astype(jnp.bfloat16), wsel,
                           (((0,), (0,)), ((), ())),
                           preferred_element_type=jnp.float32)
    t = lax.broadcasted_iota(jnp.int32, (INPUT_DIM, BT * NLANE), 1) % NLANE
    hit = (latb == t.astype(jnp.float32)) & (t <= (MAX_LAT - 1) // 2)
    oneh = jnp.where(hit, 1.0, 0.0).astype(jnp.float32)
    for b in range(BT):
        o_ref[b] = oneh[:, b * NLANE:b * NLANE + TIME_STEPS]


def kernel(x):
    B, F = x.shape
    return pl.pallas_call(
        _body,
        grid=(B // BT,),
        in_specs=[pl.BlockSpec((BT, F), lambda i: (i, 0))],
        out_specs=pl.BlockSpec((BT, F, TIME_STEPS), lambda i: (i, 0, 0)),
        out_shape=jax.ShapeDtypeStruct((B, F, TIME_STEPS), jnp.float32),
    )(x)


# speedup vs baseline: 8.2517x; 1.1415x over previous
import jax
import jax.numpy as jnp
from jax import lax
from jax.experimental import pallas as pl

INPUT_DIM = 512
TIME_STEPS = 100
MAX_LAT = 99
BT = 8
NLANE = 128


def _body(x_ref, o_ref):
    xv = x_ref[...]
    s = jax.nn.sigmoid(xv)
    latf = jnp.clip(jnp.round((1.0 - s) * float(MAX_LAT)), 0.0, float(MAX_LAT))
    bsel = lax.broadcasted_iota(jnp.int32, (BT, BT * NLANE), 1) // NLANE
    wsel = jnp.where(bsel == lax.broadcasted_iota(jnp.int32, (BT, BT * NLANE), 0),
                     1.0, 0.0).astype(jnp.bfloat16)
    latb = lax.dot_general(latf.astype(jnp.bfloat16), wsel,
                           (((0,), (0,)), ((), ())),
                           preferred_element_type=jnp.float32)
    t = lax.broadcasted_iota(jnp.int32, (INPUT_DIM, BT * NLANE), 1) % NLANE
    hit = (latb == t.astype(jnp.float32)) & (t <= (MAX_LAT - 1) // 2)
    oneh = jnp.where(hit, 1.0, 0.0).astype(jnp.float32)
    for b in range(BT):
        o_ref[b] = oneh[:, b * NLANE:(b + 1) * NLANE]


def kernel(x):
    B, F = x.shape
    y = pl.pallas_call(
        _body,
        grid=(B // BT,),
        in_specs=[pl.BlockSpec((BT, F), lambda i: (i, 0))],
        out_specs=pl.BlockSpec((BT, F, NLANE), lambda i: (i, 0, 0)),
        out_shape=jax.ShapeDtypeStruct((B, F, NLANE), jnp.float32),
    )(x)
    return y[:, :, :TIME_STEPS]
